# baseline (speedup 1.0000x reference)
import functools

import jax
import jax.numpy as jnp
from jax import lax
from jax.experimental import pallas as pl
from jax.experimental.pallas import tpu as pltpu
from jax.experimental.pallas import tpu_sc as plsc

_L = 16
_ROWS = 64
_COLS = 8192
_CHUNKS = _COLS // _L
_NW = 32
_ROWS_PER_W = _ROWS // _NW
_UNROLL = 8
_FASTC = 4

_IMIN = -(2 ** 31)
_IMAXP = 2 ** 31 - 1


def _chunk_loop(body, carry, n_chunks=_CHUNKS, unroll=_UNROLL):
    def outer(i, c):
        for u in range(unroll):
            c = body(i * (unroll * _L) + u * _L, c)
        return c
    return lax.fori_loop(0, n_chunks // unroll, outer, carry)


def _tec_body(tok_hbm, kv_hbm, out_hbm, row_v, key_v, cbuf_v, ibuf_v, kv_v):
    wid = lax.axis_index("s") * 2 + lax.axis_index("c")

    pltpu.sync_copy(kv_hbm, kv_v)
    kvec = kv_v[...]
    krv = kvec - 1

    zeros = jnp.zeros((_L,), jnp.int32)
    ones = zeros + 1
    iota = lax.iota(jnp.int32, _L)
    lane0 = iota == 0
    delta = plsc.cumsum(ones) - iota

    for r in range(_ROWS_PER_W):
        row = wid * _ROWS_PER_W + r
        pltpu.sync_copy(tok_hbm.at[row], row_v)

        def p1_body(base, cnt):
            x = row_v[pl.ds(base, _L)]
            b = lax.bitcast_convert_type(x, jnp.int32) ^ _IMIN
            ks = jnp.where(b < 0, b ^ _IMAXP, b)
            key_v[pl.ds(base, _L)] = ks
            return cnt + plsc.all_reduce_population_count(ks < 0)
        cnt1 = _chunk_loop(p1_body, zeros)
        k0 = key_v[pl.ds(0, _L)]
        natk0 = jnp.take(k0, zeros, mode="wrap")
        cnt1 = cnt1 + jnp.where(natk0 < 0, 0, 1)
        key_v[pl.ds(0, _L)] = jnp.where(lane0, _IMIN, k0)

        acc1 = cnt1 <= krv
        pu8v = jnp.where(acc1, 128, 0)
        blv = jnp.where(acc1, cnt1, 0)

        def bitpass(_, st):
            pu8, bitv, bl = st
            cand8 = pu8 | bitv
            candk = lax.shift_left(cand8 ^ 128, 24)
            def cnt_body(base, cnt):
                m = key_v[pl.ds(base, _L)] < candk
                return cnt + plsc.all_reduce_population_count(m)
            cnt = _chunk_loop(cnt_body, zeros)
            acc = cnt <= krv
            return (jnp.where(acc, cand8, pu8),
                    lax.shift_right_logical(bitv, ones),
                    jnp.where(acc, cnt, bl))
        pu8v, _, blv = lax.fori_loop(0, 7, bitpass, (pu8v, zeros + 64, blv))

        candtop = lax.shift_left(pu8v ^ 128, 24)
        krg = krv - blv

        for j in range(_FASTC):
            cbuf_v[pl.ds(j * _L, _L)] = zeros + _IMAXP
        def comp_body(base, pos):
            ks = key_v[pl.ds(base, _L)]
            m = lax.shift_right_logical(ks ^ _IMIN, 24) == pu8v
            plsc.store_compressed(cbuf_v.at[pl.ds(pos, _L)], ks, mask=m)
            plsc.store_compressed(ibuf_v.at[pl.ds(pos, _L)], iota + base,
                                  mask=m)
            return pos + plsc.all_reduce_population_count(m)[0]
        pos = _chunk_loop(comp_body, jnp.int32(0))
        cbuf_v[pl.ds(pos, _L)] = zeros + _IMAXP
        fit = pos <= _FASTC * _L
        nch = lax.shift_right_logical(pos + (_L - 1), 4)

        def bit_body(_, st):
            puv, bitv = st
            candv = candtop | puv | bitv
            cnt = zeros
            for j in range(_FASTC):
                m = cbuf_v[pl.ds(j * _L, _L)] < candv
                cnt = cnt + plsc.all_reduce_population_count(m)
            return (jnp.where(cnt <= krg, puv | bitv, puv),
                    lax.shift_right_logical(bitv, ones))
        puv, _ = lax.fori_loop(0, 24, bit_body, (zeros, zeros + (1 << 23)))
        cl = zeros
        for j in range(_FASTC):
            m = cbuf_v[pl.ds(j * _L, _L)] < (candtop | puv)
            cl = cl + plsc.all_reduce_population_count(m)

        trips = jnp.where(fit, 0, 24 * nch)
        def flat_body(_, st):
            j, puv2, bitv2, cnt, bl2 = st
            v = cbuf_v[pl.ds(j * _L, _L)]
            m = v < (candtop | puv2 | bitv2)
            cnt = cnt + plsc.all_reduce_population_count(m)
            last = j + 1 >= nch
            acc = cnt <= krg
            upd = last & acc
            puv2 = jnp.where(upd, puv2 | bitv2, puv2)
            bl2 = jnp.where(upd, cnt, bl2)
            bitv2 = jnp.where(last, lax.shift_right_logical(bitv2, ones), bitv2)
            cnt = jnp.where(last, zeros, cnt)
            j = jnp.where(last, 0, j + 1)
            return j, puv2, bitv2, cnt, bl2
        _, puv2, _, _, bl2 = lax.fori_loop(
            0, trips, flat_body,
            (jnp.int32(0), zeros, zeros + (1 << 23), zeros, zeros))

        fitv = jnp.broadcast_to(fit, (_L,))
        t_key = candtop | jnp.where(fitv, puv, puv2)
        cl = jnp.where(fitv, cl, bl2)
        quota = kvec - (blv + cl)

        def mask_body(base, c):
            ks = key_v[pl.ds(base, _L)]
            key_v[pl.ds(base, _L)] = jnp.where(ks < t_key, 1, 0)
            return c
        _chunk_loop(mask_body, zeros)

        def tie_body(j, carry):
            cb = cbuf_v[pl.ds(j * _L, _L)]
            ib = ibuf_v[pl.ds(j * _L, _L)]
            eqm = cb == t_key
            eqi = jnp.where(eqm, 1, 0)
            excl = plsc.cumsum(eqi) - eqi * delta + carry
            keep = eqm & (excl < quota)
            plsc.store_scatter(key_v, [ib], ones, mask=keep)
            return carry + plsc.all_reduce_population_count(eqm)
        lax.fori_loop(0, nch, tie_body, zeros)

        pltpu.sync_copy(key_v, out_hbm.at[row])


@jax.jit
def _select_mask(token_sequence, kvec):
    mesh = plsc.VectorSubcoreMesh(core_axis_name="c", subcore_axis_name="s")
    f = pl.kernel(
        _tec_body,
        out_type=jax.ShapeDtypeStruct((_ROWS, _COLS), jnp.int32),
        mesh=mesh,
        scratch_types=[
            pltpu.VMEM((_COLS,), jnp.float32),
            pltpu.VMEM((_COLS,), jnp.int32),
            pltpu.VMEM((_COLS + _L,), jnp.int32),
            pltpu.VMEM((_COLS + _L,), jnp.int32),
            pltpu.VMEM((_L,), jnp.int32),
        ],
        compiler_params=pltpu.CompilerParams(needs_layout_passes=False),
    )
    return f(token_sequence, kvec)


def kernel(token_sequence, embedding_sequence, compression_rate):
    seq_len = token_sequence.shape[1]
    c = compression_rate.reshape(-1)[0]
    scaled = seq_len * c
    fs = jnp.floor(scaled)
    k = jnp.where(scaled == fs, seq_len - fs, seq_len - fs - 1.0).astype(jnp.int32)
    k = jnp.maximum(k, 1)
    kvec = jnp.broadcast_to(k, (_L,)).astype(jnp.int32)
    mask = _select_mask(token_sequence, kvec)
    y = mask.astype(bool)
    return (y, y)


# speedup vs baseline: 1.6885x; 1.6885x over previous
import functools

import jax
import jax.numpy as jnp
from jax import lax
from jax.experimental import pallas as pl
from jax.experimental.pallas import tpu as pltpu
from jax.experimental.pallas import tpu_sc as plsc

_L = 16
_ROWS = 64
_COLS = 8192
_CHUNKS = _COLS // _L
_NW = 32
_ROWS_PER_W = _ROWS // _NW
_UNROLL = 8
_FASTC = 4

_IMIN = -(2 ** 31)
_IMAXP = 2 ** 31 - 1


def _chunk_loop(body, carry, n_chunks=_CHUNKS, unroll=_UNROLL):
    def outer(i, c):
        for u in range(unroll):
            c = body(i * (unroll * _L) + u * _L, c)
        return c
    return lax.fori_loop(0, n_chunks // unroll, outer, carry)


def _tec_body(tok_hbm, kv_hbm, out_hbm, flag_hbm, row_v, key_v, cbuf_v,
              ibuf_v, kv_v, fl_v):
    wid = lax.axis_index("s") * 2 + lax.axis_index("c")

    pltpu.sync_copy(kv_hbm, kv_v)
    kvec = kv_v[...]
    krv = kvec - 1

    zeros = jnp.zeros((_L,), jnp.int32)
    ones = zeros + 1
    iota = lax.iota(jnp.int32, _L)
    lane0 = iota == 0
    delta = plsc.cumsum(ones) - iota

    maxpos = jnp.int32(0)
    for r in range(_ROWS_PER_W):
        row = wid * _ROWS_PER_W + r
        pltpu.sync_copy(tok_hbm.at[row], row_v)

        def p1_body(base, cnt):
            x = row_v[pl.ds(base, _L)]
            b = lax.bitcast_convert_type(x, jnp.int32) ^ _IMIN
            ks = jnp.where(b < 0, b ^ _IMAXP, b)
            key_v[pl.ds(base, _L)] = ks
            return cnt + plsc.all_reduce_population_count(ks < 0)
        cnt1 = _chunk_loop(p1_body, zeros)
        k0 = key_v[pl.ds(0, _L)]
        natk0 = jnp.take(k0, zeros, mode="wrap")
        cnt1 = cnt1 + jnp.where(natk0 < 0, 0, 1)
        key_v[pl.ds(0, _L)] = jnp.where(lane0, _IMIN, k0)

        acc1 = cnt1 <= krv
        pu16v = jnp.where(acc1, 1 << 15, 0)
        blv = jnp.where(acc1, cnt1, 0)

        def bitpass(_, st):
            pu16, bitv, bl = st
            cand16 = pu16 | bitv
            candk = lax.shift_left(cand16 ^ (1 << 15), 16)
            def cnt_body(base, cnt):
                m = key_v[pl.ds(base, _L)] < candk
                return cnt + plsc.all_reduce_population_count(m)
            cnt = _chunk_loop(cnt_body, zeros)
            acc = cnt <= krv
            return (jnp.where(acc, cand16, pu16),
                    lax.shift_right_logical(bitv, ones),
                    jnp.where(acc, cnt, bl))
        pu16v, _, blv = lax.fori_loop(
            0, 15, bitpass, (pu16v, zeros + (1 << 14), blv))

        candtop = lax.shift_left(pu16v ^ (1 << 15), 16)
        krg = krv - blv

        for j in range(_FASTC + 1):
            cbuf_v[pl.ds(j * _L, _L)] = zeros + _IMAXP
        def comp_body(base, pos):
            ks = key_v[pl.ds(base, _L)]
            m = lax.shift_right_logical(ks ^ _IMIN, 16) == pu16v
            posc = jnp.minimum(pos, (_FASTC + 1) * _L)
            plsc.store_compressed(cbuf_v.at[pl.ds(posc, _L)], ks, mask=m)
            plsc.store_compressed(ibuf_v.at[pl.ds(posc, _L)], iota + base,
                                  mask=m)
            return pos + plsc.all_reduce_population_count(m)[0]
        pos = _chunk_loop(comp_body, jnp.int32(0))
        maxpos = jnp.maximum(maxpos, pos)

        def bit_body(_, st):
            puv, bitv = st
            candv = candtop | puv | bitv
            cnt = zeros
            for j in range(_FASTC):
                m = cbuf_v[pl.ds(j * _L, _L)] < candv
                cnt = cnt + plsc.all_reduce_population_count(m)
            return (jnp.where(cnt <= krg, puv | bitv, puv),
                    lax.shift_right_logical(bitv, ones))
        puv, _ = lax.fori_loop(0, 24, bit_body, (zeros, zeros + (1 << 23)))
        t_key = candtop | puv
        cl = zeros
        for j in range(_FASTC):
            m = cbuf_v[pl.ds(j * _L, _L)] < t_key
            cl = cl + plsc.all_reduce_population_count(m)
        quota = kvec - (blv + cl)

        def mask_body(base, c):
            ks = key_v[pl.ds(base, _L)]
            key_v[pl.ds(base, _L)] = jnp.where(ks < t_key, 1, 0)
            return c
        _chunk_loop(mask_body, zeros)

        carry = zeros
        for j in range(_FASTC):
            cb = cbuf_v[pl.ds(j * _L, _L)]
            ib = ibuf_v[pl.ds(j * _L, _L)]
            eqm = cb == t_key
            eqi = jnp.where(eqm, 1, 0)
            excl = plsc.cumsum(eqi) - eqi * delta + carry
            keep = eqm & (excl < quota)
            plsc.store_scatter(key_v, [ib], ones, mask=keep)
            carry = carry + plsc.all_reduce_population_count(eqm)

        pltpu.sync_copy(key_v, out_hbm.at[row])

    fl_v[...] = jnp.broadcast_to(maxpos, (_L,))
    pltpu.sync_copy(fl_v, flag_hbm.at[wid])


def _tec_body_full(tok_hbm, kv_hbm, out_hbm, row_v, key_v, kv_v):
    wid = lax.axis_index("s") * 2 + lax.axis_index("c")

    pltpu.sync_copy(kv_hbm, kv_v)
    kvec = kv_v[...]
    km1 = kvec - 1

    zeros = jnp.zeros((_L,), jnp.int32)
    ones = zeros + 1
    iota = lax.iota(jnp.int32, _L)
    delta = plsc.cumsum(ones) - iota

    for r in range(_ROWS_PER_W):
        row = wid * _ROWS_PER_W + r
        pltpu.sync_copy(tok_hbm.at[row], row_v)

        def key_body(base, c):
            x = row_v[pl.ds(base, _L)]
            b = lax.bitcast_convert_type(x, jnp.int32) ^ _IMIN
            ks = jnp.where(b < 0, b ^ _IMAXP, b)
            key_v[pl.ds(base, _L)] = ks
            return c
        _chunk_loop(key_body, zeros)
        k0 = key_v[pl.ds(0, _L)]
        key_v[pl.ds(0, _L)] = jnp.where(iota == 0, _IMIN, k0)

        def bit_body(_, st):
            pu, bit = st
            cand_u = pu | bit
            cand = cand_u ^ _IMIN
            def cnt_body(base, cnt):
                m = key_v[pl.ds(base, _L)] < cand
                return cnt + plsc.all_reduce_population_count(m)
            cnt = _chunk_loop(cnt_body, zeros)
            take = cnt <= km1
            return jnp.where(take, cand_u, pu), lax.shift_right_logical(bit, ones)
        pu, _ = lax.fori_loop(0, 32, bit_body, (zeros, zeros + _IMIN))
        t_key = pu ^ _IMIN

        def less_body(base, cnt):
            m = key_v[pl.ds(base, _L)] < t_key
            return cnt + plsc.all_reduce_population_count(m)
        count_less = _chunk_loop(less_body, zeros)
        quota = kvec - count_less

        def mask_body(base, carry):
            c = key_v[pl.ds(base, _L)]
            ltm = c < t_key
            eqm = c == t_key
            eqi = jnp.where(eqm, 1, 0)
            excl = plsc.cumsum(eqi) - eqi * delta + carry
            keep = ltm | (eqm & (excl < quota))
            key_v[pl.ds(base, _L)] = jnp.where(keep, 1, 0)
            return carry + plsc.all_reduce_population_count(eqm)
        _chunk_loop(mask_body, zeros)

        pltpu.sync_copy(key_v, out_hbm.at[row])


def _fast_call(token_sequence, kvec):
    mesh = plsc.VectorSubcoreMesh(core_axis_name="c", subcore_axis_name="s")
    f = pl.kernel(
        _tec_body,
        out_type=(jax.ShapeDtypeStruct((_ROWS, _COLS), jnp.int32),
                  jax.ShapeDtypeStruct((_NW, _L), jnp.int32)),
        mesh=mesh,
        scratch_types=[
            pltpu.VMEM((_COLS,), jnp.float32),
            pltpu.VMEM((_COLS,), jnp.int32),
            pltpu.VMEM(((_FASTC + 2) * _L,), jnp.int32),
            pltpu.VMEM(((_FASTC + 2) * _L,), jnp.int32),
            pltpu.VMEM((_L,), jnp.int32),
            pltpu.VMEM((_L,), jnp.int32),
        ],
        compiler_params=pltpu.CompilerParams(needs_layout_passes=False),
    )
    return f(token_sequence, kvec)


def _full_call(token_sequence, kvec):
    mesh = plsc.VectorSubcoreMesh(core_axis_name="c", subcore_axis_name="s")
    f = pl.kernel(
        _tec_body_full,
        out_type=jax.ShapeDtypeStruct((_ROWS, _COLS), jnp.int32),
        mesh=mesh,
        scratch_types=[
            pltpu.VMEM((_COLS,), jnp.float32),
            pltpu.VMEM((_COLS,), jnp.int32),
            pltpu.VMEM((_L,), jnp.int32),
        ],
        compiler_params=pltpu.CompilerParams(needs_layout_passes=False),
    )
    return f(token_sequence, kvec)


@jax.jit
def _select_mask(token_sequence, kvec):
    mask, flags = _fast_call(token_sequence, kvec)
    fit = jnp.all(flags <= _FASTC * _L)
    return lax.cond(fit,
                    lambda: mask,
                    lambda: _full_call(token_sequence, kvec))


def kernel(token_sequence, embedding_sequence, compression_rate):
    seq_len = token_sequence.shape[1]
    c = compression_rate.reshape(-1)[0]
    scaled = seq_len * c
    fs = jnp.floor(scaled)
    k = jnp.where(scaled == fs, seq_len - fs, seq_len - fs - 1.0).astype(jnp.int32)
    k = jnp.maximum(k, 1)
    kvec = jnp.broadcast_to(k, (_L,)).astype(jnp.int32)
    mask = _select_mask(token_sequence, kvec)
    y = mask.astype(bool)
    return (y, y)


# speedup vs baseline: 1.7831x; 1.0560x over previous
import functools

import jax
import jax.numpy as jnp
from jax import lax
from jax.experimental import pallas as pl
from jax.experimental.pallas import tpu as pltpu
from jax.experimental.pallas import tpu_sc as plsc

_L = 16
_ROWS = 64
_COLS = 8192
_CHUNKS = _COLS // _L
_NW = 32
_ROWS_PER_W = _ROWS // _NW
_UNROLL = 8

_IMIN = -(2 ** 31)
_IMAXP = 2 ** 31 - 1


def _chunk_loop(body, carry):
    def outer(i, c):
        for u in range(_UNROLL):
            c = body(i * (_UNROLL * _L) + u * _L, c)
        return c
    return lax.fori_loop(0, _CHUNKS // _UNROLL, outer, carry)


def _tec_body(tok_hbm, kv_hbm, out_hbm, row_v, key_v, mask_v, kv_v):
    wid = lax.axis_index("s") * 2 + lax.axis_index("c")

    pltpu.sync_copy(kv_hbm, kv_v)
    kvec = kv_v[...]
    km1 = kvec - 1

    zeros = jnp.zeros((_L,), jnp.int32)
    ones = zeros + 1
    iota = lax.iota(jnp.int32, _L)
    delta = plsc.cumsum(ones) - iota

    for r in range(_ROWS_PER_W):
        row = wid * _ROWS_PER_W + r
        pltpu.sync_copy(tok_hbm.at[row], row_v)

        def key_body(base, c):
            x = row_v[pl.ds(base, _L)]
            b = lax.bitcast_convert_type(x, jnp.int32) ^ _IMIN
            ks = jnp.where(b < 0, b ^ _IMAXP, b)
            key_v[pl.ds(base, _L)] = ks
            return c
        _chunk_loop(key_body, zeros)
        k0 = key_v[pl.ds(0, _L)]
        key_v[pl.ds(0, _L)] = jnp.where(iota == 0, _IMIN, k0)

        def bit_body(_, st):
            pu, bit = st
            cand_u = pu | bit
            cand = cand_u ^ _IMIN
            def cnt_body(base, cnt):
                m = key_v[pl.ds(base, _L)] < cand
                return cnt + plsc.all_reduce_population_count(m)
            cnt = _chunk_loop(cnt_body, zeros)
            take = cnt <= km1
            return jnp.where(take, cand_u, pu), lax.shift_right_logical(bit, ones)
        pu, _ = lax.fori_loop(0, 32, bit_body, (zeros, zeros + _IMIN))
        t_key = pu ^ _IMIN

        def less_body(base, cnt):
            m = key_v[pl.ds(base, _L)] < t_key
            return cnt + plsc.all_reduce_population_count(m)
        count_less = _chunk_loop(less_body, zeros)
        quota = kvec - count_less

        def mask_body(base, carry):
            c = key_v[pl.ds(base, _L)]
            ltm = c < t_key
            eqm = c == t_key
            eqi = jnp.where(eqm, 1, 0)
            excl = plsc.cumsum(eqi) - eqi * delta + carry
            keep = ltm | (eqm & (excl < quota))
            mask_v[pl.ds(base, _L)] = jnp.where(keep, 1, 0)
            return carry + plsc.all_reduce_population_count(eqm)
        _chunk_loop(mask_body, zeros)

        pltpu.sync_copy(mask_v, out_hbm.at[row])


@jax.jit
def _select_mask(token_sequence, kvec):
    mesh = plsc.VectorSubcoreMesh(core_axis_name="c", subcore_axis_name="s")
    f = pl.kernel(
        _tec_body,
        out_type=jax.ShapeDtypeStruct((_ROWS, _COLS), jnp.int32),
        mesh=mesh,
        scratch_types=[
            pltpu.VMEM((_COLS,), jnp.float32),
            pltpu.VMEM((_COLS,), jnp.int32),
            pltpu.VMEM((_COLS,), jnp.int32),
            pltpu.VMEM((_L,), jnp.int32),
        ],
        compiler_params=pltpu.CompilerParams(needs_layout_passes=False),
    )
    return f(token_sequence, kvec)


def kernel(token_sequence, embedding_sequence, compression_rate):
    seq_len = token_sequence.shape[1]
    c = compression_rate.reshape(-1)[0]
    scaled = seq_len * c
    fs = jnp.floor(scaled)
    k = jnp.where(scaled == fs, seq_len - fs, seq_len - fs - 1.0).astype(jnp.int32)
    k = jnp.maximum(k, 1)
    kvec = jnp.broadcast_to(k, (_L,)).astype(jnp.int32)
    mask = _select_mask(token_sequence, kvec)
    y = mask.astype(bool)
    return (y, y)


# speedup vs baseline: 1.8044x; 1.0120x over previous
import functools

import jax
import jax.numpy as jnp
from jax import lax
from jax.experimental import pallas as pl
from jax.experimental.pallas import tpu as pltpu
from jax.experimental.pallas import tpu_sc as plsc

_L = 16
_ROWS = 64
_COLS = 8192
_CHUNKS = _COLS // _L
_NW = 32
_ROWS_PER_W = _ROWS // _NW
_UNROLL = 8

_IMIN = -(2 ** 31)
_IMAXP = 2 ** 31 - 1


def _chunk_loop(body, carry):
    def outer(i, c):
        for u in range(_UNROLL):
            c = body(i * (_UNROLL * _L) + u * _L, c)
        return c
    return lax.fori_loop(0, _CHUNKS // _UNROLL, outer, carry)


def _tec_body(tok_hbm, kv_hbm, out_hbm, row_v, key_v, mask_v, kv_v):
    wid = lax.axis_index("s") * 2 + lax.axis_index("c")

    pltpu.sync_copy(kv_hbm, kv_v)
    kvec = kv_v[...]
    km1 = kvec - 1

    zeros = jnp.zeros((_L,), jnp.int32)
    ones = zeros + 1
    iota = lax.iota(jnp.int32, _L)
    delta = plsc.cumsum(ones) - iota

    for r in range(_ROWS_PER_W):
        row = wid * _ROWS_PER_W + r
        pltpu.sync_copy(tok_hbm.at[row], row_v)

        def key_body(base, c):
            x = row_v[pl.ds(base, _L)]
            b = lax.bitcast_convert_type(x, jnp.int32) ^ _IMIN
            ks = jnp.where(b < 0, b ^ _IMAXP, b)
            key_v[pl.ds(base, _L)] = ks
            return c
        _chunk_loop(key_body, zeros)
        k0 = key_v[pl.ds(0, _L)]
        key_v[pl.ds(0, _L)] = jnp.where(iota == 0, _IMIN, k0)

        def bit_body(_, st):
            pu, bit, bl = st
            cand_u = pu | bit
            cand = cand_u ^ _IMIN
            def cnt_body(base, cnt):
                m = key_v[pl.ds(base, _L)] < cand
                return cnt + plsc.all_reduce_population_count(m)
            cnt = _chunk_loop(cnt_body, zeros)
            take = cnt <= km1
            return (jnp.where(take, cand_u, pu),
                    lax.shift_right_logical(bit, ones),
                    jnp.where(take, cnt, bl))
        pu, _, count_less = lax.fori_loop(
            0, 32, bit_body, (zeros, zeros + _IMIN, zeros))
        t_key = pu ^ _IMIN
        quota = kvec - count_less

        def mask_body(base, carry):
            c = key_v[pl.ds(base, _L)]
            ltm = c < t_key
            eqm = c == t_key
            eqi = jnp.where(eqm, 1, 0)
            excl = plsc.cumsum(eqi) - eqi * delta + carry
            keep = ltm | (eqm & (excl < quota))
            mask_v[pl.ds(base, _L)] = jnp.where(keep, 1, 0)
            return carry + plsc.all_reduce_population_count(eqm)
        _chunk_loop(mask_body, zeros)

        pltpu.sync_copy(mask_v, out_hbm.at[row])


@jax.jit
def _select_mask(token_sequence, kvec):
    mesh = plsc.VectorSubcoreMesh(core_axis_name="c", subcore_axis_name="s")
    f = pl.kernel(
        _tec_body,
        out_type=jax.ShapeDtypeStruct((_ROWS, _COLS), jnp.int32),
        mesh=mesh,
        scratch_types=[
            pltpu.VMEM((_COLS,), jnp.float32),
            pltpu.VMEM((_COLS,), jnp.int32),
            pltpu.VMEM((_COLS,), jnp.int32),
            pltpu.VMEM((_L,), jnp.int32),
        ],
        compiler_params=pltpu.CompilerParams(needs_layout_passes=False),
    )
    return f(token_sequence, kvec)


def kernel(token_sequence, embedding_sequence, compression_rate):
    seq_len = token_sequence.shape[1]
    c = compression_rate.reshape(-1)[0]
    scaled = seq_len * c
    fs = jnp.floor(scaled)
    k = jnp.where(scaled == fs, seq_len - fs, seq_len - fs - 1.0).astype(jnp.int32)
    k = jnp.maximum(k, 1)
    kvec = jnp.broadcast_to(k, (_L,)).astype(jnp.int32)
    mask = _select_mask(token_sequence, kvec)
    y = mask.astype(bool)
    return (y, y)
